# baseline (speedup 1.0000x reference)
import functools

import jax
import jax.numpy as jnp
import numpy as np
from jax import lax
from jax.experimental import pallas as pl
from jax.experimental.pallas import tpu as pltpu
from jax.experimental.pallas import tpu_sc as plsc

B, H, W, C = 2, 224, 224, 96
HW = H * W
N_PLANES = B * C
RT = 8
MROWS = RT * W
KDIM = 9 * C
NOUT = 2 * C
PAD = 256
CH = 3584
N_CHUNKS = HW // CH
HALF_CHUNKS = N_CHUNKS // 2
ITERS = CH // 16


def _grid_base() -> np.ndarray:
    lq = np.arange(MROWS)
    ch = np.arange(NOUT)
    h = lq // 2
    even = (lq % 2 == 0)
    base_y = (h // W)[:, None] + 112 * (ch[None, :] % 2)
    base_x = (h % W)[:, None] + np.zeros((1, NOUT), np.int64)
    return np.where(even[:, None], base_y, base_x).astype(np.float32)


_GRID_BASE = _grid_base()


RRT = 8
HPAD = 29 * RRT


def _retile_body(xa_ref, xb_ref, o_ref):
    r = pl.program_id(1)
    zrow = jnp.zeros((1, C), jnp.float32)
    for j in range(RRT):
        img_row = r * RRT + j - 1
        src = xa_ref[0, RRT - 1] if j == 0 else xb_ref[0, j - 1]
        valid = jnp.logical_and(img_row >= 0, img_row < H)
        scale = jnp.where(valid, jnp.float32(1), jnp.float32(0))
        o_ref[0, j, pl.ds(1, W), :] = src.T * scale
        o_ref[0, j, pl.ds(0, 1), :] = zrow
        o_ref[0, j, pl.ds(W + 1, 1), :] = zrow


def _retile(xt, bi):
    nrb = H // RRT - 1
    return pl.pallas_call(
        _retile_body,
        grid=(1, HPAD // RRT),
        in_specs=[
            pl.BlockSpec((1, RRT, C, W),
                         lambda b, r, bi=bi: (bi, jnp.clip(r - 1, 0, nrb), 0, 0)),
            pl.BlockSpec((1, RRT, C, W),
                         lambda b, r, bi=bi: (bi, jnp.clip(r, 0, nrb), 0, 0)),
        ],
        out_specs=pl.BlockSpec((1, RRT, W + 2, C), lambda b, r: (b, r, 0, 0)),
        out_shape=jax.ShapeDtypeStruct((1, HPAD, W + 2, C), jnp.float32),
    )(xt, xt)


def _conv_body(x_ref, w_ref, b_ref, g_ref, co_ref, xp_ref):
    r = pl.program_id(1)
    i0 = r * RT
    parts = []
    for ki in range(3):
        for kj in range(3):
            xs = x_ref[0, pl.ds(i0 + ki, RT), pl.ds(kj, W), :]
            parts.append(xs.reshape(MROWS, C))
    patch = jnp.concatenate(parts, axis=-1)
    w2d = w_ref[...].reshape(KDIM, NOUT)
    acc = jnp.dot(patch, w2d, preferred_element_type=jnp.float32)
    center = parts[4]
    acc = acc + b_ref[...] + g_ref[...]
    row_par = lax.broadcasted_iota(jnp.int32, (MROWS, NOUT), 0) & 1
    radd = jnp.where(row_par == 0,
                     jnp.float32(RT // 2) * r.astype(jnp.float32),
                     jnp.float32(0))
    coords = jnp.minimum(jnp.maximum(acc + radd, 0.0), jnp.float32(W - 1))
    co_ref[0] = coords.T
    xp_ref[0] = center.T


def _conv_coords(xpad, w4d, b2d):
    return pl.pallas_call(
        _conv_body,
        grid=(1, H // RT),
        in_specs=[
            pl.BlockSpec((1, HPAD, W + 2, C), lambda b, r: (0, 0, 0, 0)),
            pl.BlockSpec((3, 3, C, NOUT), lambda b, r: (0, 0, 0, 0)),
            pl.BlockSpec((1, NOUT), lambda b, r: (0, 0)),
            pl.BlockSpec((MROWS, NOUT), lambda b, r: (0, 0)),
        ],
        out_specs=[
            pl.BlockSpec((1, NOUT, MROWS), lambda b, r: (b, 0, r)),
            pl.BlockSpec((1, C, MROWS), lambda b, r: (b, 0, r)),
        ],
        out_shape=[
            jax.ShapeDtypeStruct((1, NOUT, HW), jnp.float32),
            jax.ShapeDtypeStruct((1, C, HW), jnp.float32),
        ],
        compiler_params=pltpu.CompilerParams(
            vmem_limit_bytes=100 * 1024 * 1024),
    )(xpad, w4d, b2d, jnp.asarray(_GRID_BASE))


def _sc_sample_body(xp_hbm, co_hbm, out_hbm, pv0, pv1,
                    ob0, ob1, cb0, cb1, si0, si1, so0, so1, sp0, sp1):
    info = plsc.get_sparse_core_info()
    nc = info.num_cores
    wid = lax.axis_index("s") * nc + lax.axis_index("c")
    n_pl = xp_hbm.shape[0] * xp_hbm.shape[1]
    planes_per = n_pl // (nc * info.num_subcores)
    cbufs, obufs = [cb0, cb1], [ob0, ob1]
    sins, souts = [si0, si1], [so0, so1]
    pvs, sps = [pv0, pv1], [sp0, sp1]

    zero16 = jnp.zeros((16,), jnp.float32)
    for z in range(PAD // 16):
        pv0[pl.ds(HW + z * 16, 16)] = zero16
        pv1[pl.ds(HW + z * 16, 16)] = zero16

    def bm(pi):
        plane = wid * planes_per + pi
        b = lax.div(plane, jnp.int32(C))
        return b, plane - b * C

    def pin(pi):
        b, m = bm(pi)
        k = pi % 2
        return pltpu.make_async_copy(
            xp_hbm.at[b, m], pvs[k].at[pl.ds(0, HW)], sps[k])

    pin(0).start()
    for pi in range(planes_per):
        b, m = bm(pi)
        plane_v = pvs[pi % 2]

        def cin(ci, b=b, m=m):
            k = ci % 2
            half = ci // HALF_CHUNKS
            src = 2 * ci * CH - half * HW
            return pltpu.make_async_copy(
                co_hbm.at[b, 2 * m + half, pl.ds(src, 2 * CH)],
                cbufs[k], sins[k])

        def cout(ci, b=b, m=m):
            k = ci % 2
            return pltpu.make_async_copy(
                obufs[k], out_hbm.at[b, m, pl.ds(ci * CH, CH)], souts[k])

        cin(0).start()
        pin(pi).wait()
        if pi + 1 < planes_per:
            pin(pi + 1).start()
        for ci in range(N_CHUNKS):
            cur = ci % 2
            if ci + 1 < N_CHUNKS:
                cin(ci + 1).start()
            cin(ci).wait()
            if ci >= 2:
                cout(ci - 2).wait()
            cbuf, obuf = cbufs[cur], obufs[cur]

            @plsc.parallel_loop(0, ITERS, 1, unroll=4)
            def body(i, cbuf=cbuf, obuf=obuf, plane_v=plane_v):
                off = i * 16
                rel = off + lax.iota(jnp.int32, 16)
                cy = plsc.load_gather(cbuf, [2 * rel])
                cx = plsc.load_gather(cbuf, [2 * rel + 1])
                y0 = cy.astype(jnp.int32)
                x0 = cx.astype(jnp.int32)
                fy = cy - y0.astype(jnp.float32)
                fx = cx - x0.astype(jnp.float32)
                idx = y0 * W + x0
                v00 = plsc.load_gather(plane_v, [idx])
                v01 = plsc.load_gather(plane_v, [idx + 1])
                v10 = plsc.load_gather(plane_v, [idx + W])
                v11 = plsc.load_gather(plane_v, [idx + W + 1])
                top = v00 + (v10 - v00) * fy
                bot = v01 + (v11 - v01) * fy
                obuf[pl.ds(off, 16)] = top + (bot - top) * fx

            cout(ci).start()
        cout(N_CHUNKS - 2).wait()
        cout(N_CHUNKS - 1).wait()


@functools.cache
def _sc_sample():
    return functools.partial(
        pl.kernel,
        mesh=plsc.VectorSubcoreMesh(core_axis_name="c", subcore_axis_name="s"),
        out_type=jax.ShapeDtypeStruct((1, C, HW), jnp.float32),
        compiler_params=pltpu.CompilerParams(needs_layout_passes=False),
        scratch_types=[
            pltpu.VMEM((HW + PAD,), jnp.float32),
            pltpu.VMEM((HW + PAD,), jnp.float32),
            pltpu.VMEM((CH,), jnp.float32),
            pltpu.VMEM((CH,), jnp.float32),
            pltpu.VMEM((2 * CH,), jnp.float32),
            pltpu.VMEM((2 * CH,), jnp.float32),
            pltpu.SemaphoreType.DMA,
            pltpu.SemaphoreType.DMA,
            pltpu.SemaphoreType.DMA,
            pltpu.SemaphoreType.DMA,
            pltpu.SemaphoreType.DMA,
            pltpu.SemaphoreType.DMA,
        ],
    )(_sc_sample_body)


def _untrans0_body(p_ref, o_ref):
    o_ref[0] = p_ref[0].T.reshape(RT, W, C)


def _untrans1_body(p_ref, prev_ref, o_ref):
    del prev_ref
    o_ref[0] = p_ref[0].T.reshape(RT, W, C)


def _untranspose0(p0):
    return pl.pallas_call(
        _untrans0_body,
        grid=(HW // MROWS,),
        in_specs=[pl.BlockSpec((1, C, MROWS), lambda r: (0, 0, r))],
        out_specs=pl.BlockSpec((1, RT, W, C), lambda r: (0, r, 0, 0)),
        out_shape=jax.ShapeDtypeStruct((B, H, W, C), jnp.float32),
    )(p0)


def _untranspose1(p1, prev):
    return pl.pallas_call(
        _untrans1_body,
        grid=(HW // MROWS,),
        in_specs=[
            pl.BlockSpec((1, C, MROWS), lambda r: (0, 0, r)),
            pl.BlockSpec(memory_space=pl.ANY),
        ],
        out_specs=pl.BlockSpec((1, RT, W, C), lambda r: (1, r, 0, 0)),
        out_shape=jax.ShapeDtypeStruct((B, H, W, C), jnp.float32),
        input_output_aliases={1: 0},
    )(p1, prev)


def kernel(x, W_conv, b_conv):
    xt = jnp.transpose(x, (0, 1, 3, 2))
    b2d = b_conv.reshape(1, NOUT)
    outs = []
    for bi in range(B):
        xpad = _retile(xt, bi)
        coords, xplanes = _conv_coords(xpad, W_conv, b2d)
        outs.append(_sc_sample()(xplanes, coords))
    nhwc = _untranspose0(outs[0])
    return _untranspose1(outs[1], nhwc)


# speedup vs baseline: 1.0053x; 1.0053x over previous
import functools

import jax
import jax.numpy as jnp
import numpy as np
from jax import lax
from jax.experimental import pallas as pl
from jax.experimental.pallas import tpu as pltpu
from jax.experimental.pallas import tpu_sc as plsc

B, H, W, C = 2, 224, 224, 96
HW = H * W
N_PLANES = B * C
RT = 8
MROWS = RT * W
KDIM = 9 * C
NOUT = 2 * C
PAD = 256
CH = 6272
N_CHUNKS = HW // CH
HALF_CHUNKS = N_CHUNKS // 2
ITERS = CH // 16


def _grid_base() -> np.ndarray:
    lq = np.arange(MROWS)
    ch = np.arange(NOUT)
    h = lq // 2
    even = (lq % 2 == 0)
    base_y = (h // W)[:, None] + 112 * (ch[None, :] % 2)
    base_x = (h % W)[:, None] + np.zeros((1, NOUT), np.int64)
    return np.where(even[:, None], base_y, base_x).astype(np.float32)


_GRID_BASE = _grid_base()


RRT = 8
HPAD = 29 * RRT


def _retile_body(xa_ref, xb_ref, o_ref):
    r = pl.program_id(1)
    zrow = jnp.zeros((1, C), jnp.float32)
    for j in range(RRT):
        img_row = r * RRT + j - 1
        src = xa_ref[0, RRT - 1] if j == 0 else xb_ref[0, j - 1]
        valid = jnp.logical_and(img_row >= 0, img_row < H)
        scale = jnp.where(valid, jnp.float32(1), jnp.float32(0))
        o_ref[0, j, pl.ds(1, W), :] = src.T * scale
        o_ref[0, j, pl.ds(0, 1), :] = zrow
        o_ref[0, j, pl.ds(W + 1, 1), :] = zrow


def _retile(xt, bi):
    nrb = H // RRT - 1
    return pl.pallas_call(
        _retile_body,
        grid=(1, HPAD // RRT),
        in_specs=[
            pl.BlockSpec((1, RRT, C, W),
                         lambda b, r, bi=bi: (bi, jnp.clip(r - 1, 0, nrb), 0, 0)),
            pl.BlockSpec((1, RRT, C, W),
                         lambda b, r, bi=bi: (bi, jnp.clip(r, 0, nrb), 0, 0)),
        ],
        out_specs=pl.BlockSpec((1, RRT, W + 2, C), lambda b, r: (b, r, 0, 0)),
        out_shape=jax.ShapeDtypeStruct((1, HPAD, W + 2, C), jnp.float32),
    )(xt, xt)


def _conv_body(x_ref, w_ref, b_ref, g_ref, co_ref, xp_ref):
    r = pl.program_id(1)
    i0 = r * RT
    parts = []
    for ki in range(3):
        for kj in range(3):
            xs = x_ref[0, pl.ds(i0 + ki, RT), pl.ds(kj, W), :]
            parts.append(xs.reshape(MROWS, C))
    patch = jnp.concatenate(parts, axis=-1)
    w2d = w_ref[...].reshape(KDIM, NOUT)
    acc = jnp.dot(patch, w2d, preferred_element_type=jnp.float32)
    center = parts[4]
    acc = acc + b_ref[...] + g_ref[...]
    row_par = lax.broadcasted_iota(jnp.int32, (MROWS, NOUT), 0) & 1
    radd = jnp.where(row_par == 0,
                     jnp.float32(RT // 2) * r.astype(jnp.float32),
                     jnp.float32(0))
    coords = jnp.minimum(jnp.maximum(acc + radd, 0.0), jnp.float32(W - 1))
    co_ref[0] = coords.T
    xp_ref[0] = center.T


def _conv_coords(xpad, w4d, b2d):
    return pl.pallas_call(
        _conv_body,
        grid=(1, H // RT),
        in_specs=[
            pl.BlockSpec((1, HPAD, W + 2, C), lambda b, r: (0, 0, 0, 0)),
            pl.BlockSpec((3, 3, C, NOUT), lambda b, r: (0, 0, 0, 0)),
            pl.BlockSpec((1, NOUT), lambda b, r: (0, 0)),
            pl.BlockSpec((MROWS, NOUT), lambda b, r: (0, 0)),
        ],
        out_specs=[
            pl.BlockSpec((1, NOUT, MROWS), lambda b, r: (b, 0, r)),
            pl.BlockSpec((1, C, MROWS), lambda b, r: (b, 0, r)),
        ],
        out_shape=[
            jax.ShapeDtypeStruct((1, NOUT, HW), jnp.float32),
            jax.ShapeDtypeStruct((1, C, HW), jnp.float32),
        ],
        compiler_params=pltpu.CompilerParams(
            vmem_limit_bytes=100 * 1024 * 1024),
    )(xpad, w4d, b2d, jnp.asarray(_GRID_BASE))


def _sc_sample_body(xp_hbm, co_hbm, out_hbm, plane_v,
                    ob0, ob1, cb0, cb1, si0, si1, so0, so1):
    info = plsc.get_sparse_core_info()
    nc = info.num_cores
    wid = lax.axis_index("s") * nc + lax.axis_index("c")
    n_pl = xp_hbm.shape[0] * xp_hbm.shape[1]
    planes_per = n_pl // (nc * info.num_subcores)
    cbufs, obufs = [cb0, cb1], [ob0, ob1]
    sins, souts = [si0, si1], [so0, so1]

    zero16 = jnp.zeros((16,), jnp.float32)
    for z in range(PAD // 16):
        plane_v[pl.ds(HW + z * 16, 16)] = zero16

    for pi in range(planes_per):
        plane = wid * planes_per + pi
        b = lax.div(plane, jnp.int32(C))
        m = plane - b * C

        def cin(ci, b=b, m=m):
            k = ci % 2
            half = ci // HALF_CHUNKS
            src = 2 * ci * CH - half * HW
            return pltpu.make_async_copy(
                co_hbm.at[b, 2 * m + half, pl.ds(src, 2 * CH)],
                cbufs[k], sins[k])

        def cout(ci, b=b, m=m):
            k = ci % 2
            return pltpu.make_async_copy(
                obufs[k], out_hbm.at[b, m, pl.ds(ci * CH, CH)], souts[k])

        cin(0).start()
        pltpu.sync_copy(xp_hbm.at[b, m], plane_v.at[pl.ds(0, HW)])
        for ci in range(N_CHUNKS):
            cur = ci % 2
            if ci + 1 < N_CHUNKS:
                cin(ci + 1).start()
            cin(ci).wait()
            if ci >= 2:
                cout(ci - 2).wait()
            cbuf, obuf = cbufs[cur], obufs[cur]

            @plsc.parallel_loop(0, ITERS, 1, unroll=4)
            def body(i, cbuf=cbuf, obuf=obuf):
                off = i * 16
                rel = off + lax.iota(jnp.int32, 16)
                cy = plsc.load_gather(cbuf, [2 * rel])
                cx = plsc.load_gather(cbuf, [2 * rel + 1])
                y0 = cy.astype(jnp.int32)
                x0 = cx.astype(jnp.int32)
                fy = cy - y0.astype(jnp.float32)
                fx = cx - x0.astype(jnp.float32)
                idx = y0 * W + x0
                v00 = plsc.load_gather(plane_v, [idx])
                v01 = plsc.load_gather(plane_v, [idx + 1])
                v10 = plsc.load_gather(plane_v, [idx + W])
                v11 = plsc.load_gather(plane_v, [idx + W + 1])
                top = v00 + (v10 - v00) * fy
                bot = v01 + (v11 - v01) * fy
                obuf[pl.ds(off, 16)] = top + (bot - top) * fx

            cout(ci).start()
        cout(N_CHUNKS - 2).wait()
        cout(N_CHUNKS - 1).wait()


@functools.cache
def _sc_sample():
    return functools.partial(
        pl.kernel,
        mesh=plsc.VectorSubcoreMesh(core_axis_name="c", subcore_axis_name="s"),
        out_type=jax.ShapeDtypeStruct((1, C, HW), jnp.float32),
        compiler_params=pltpu.CompilerParams(needs_layout_passes=False),
        scratch_types=[
            pltpu.VMEM((HW + PAD,), jnp.float32),
            pltpu.VMEM((CH,), jnp.float32),
            pltpu.VMEM((CH,), jnp.float32),
            pltpu.VMEM((2 * CH,), jnp.float32),
            pltpu.VMEM((2 * CH,), jnp.float32),
            pltpu.SemaphoreType.DMA,
            pltpu.SemaphoreType.DMA,
            pltpu.SemaphoreType.DMA,
            pltpu.SemaphoreType.DMA,
        ],
    )(_sc_sample_body)


def _untrans0_body(p_ref, o_ref):
    o_ref[0] = p_ref[0].T.reshape(RT, W, C)


def _untrans1_body(p_ref, prev_ref, o_ref):
    del prev_ref
    o_ref[0] = p_ref[0].T.reshape(RT, W, C)


def _untranspose0(p0):
    return pl.pallas_call(
        _untrans0_body,
        grid=(HW // MROWS,),
        in_specs=[pl.BlockSpec((1, C, MROWS), lambda r: (0, 0, r))],
        out_specs=pl.BlockSpec((1, RT, W, C), lambda r: (0, r, 0, 0)),
        out_shape=jax.ShapeDtypeStruct((B, H, W, C), jnp.float32),
    )(p0)


def _untranspose1(p1, prev):
    return pl.pallas_call(
        _untrans1_body,
        grid=(HW // MROWS,),
        in_specs=[
            pl.BlockSpec((1, C, MROWS), lambda r: (0, 0, r)),
            pl.BlockSpec(memory_space=pl.ANY),
        ],
        out_specs=pl.BlockSpec((1, RT, W, C), lambda r: (1, r, 0, 0)),
        out_shape=jax.ShapeDtypeStruct((B, H, W, C), jnp.float32),
        input_output_aliases={1: 0},
    )(p1, prev)


def kernel(x, W_conv, b_conv):
    xt = jnp.transpose(x, (0, 1, 3, 2))
    b2d = b_conv.reshape(1, NOUT)
    outs = []
    for bi in range(B):
        xpad = _retile(xt, bi)
        coords, xplanes = _conv_coords(xpad, W_conv, b2d)
        outs.append(_sc_sample()(xplanes, coords))
    nhwc = _untranspose0(outs[0])
    return _untranspose1(outs[1], nhwc)
